# speedup vs baseline: 5.7527x; 5.7527x over previous
import functools
import math

import jax
import jax.numpy as jnp
from jax import lax
from jax.experimental import pallas as pl
from jax.experimental.pallas import tpu as pltpu
from jax.experimental.pallas import tpu_sc as plsc

D_MODEL = 128
SCALE = math.sqrt(D_MODEL)

NC = 2
NS = 16
NW = NC * NS

CHUNK = 128


def _scale_body(lut_ref, out_ref):
    out_ref[...] = lut_ref[...] * SCALE


def _scale_table(lut):
    v, d = lut.shape
    blk = 2000
    grid = v // blk
    return pl.pallas_call(
        _scale_body,
        grid=(grid,),
        in_specs=[pl.BlockSpec((blk, d), lambda i: (i, 0))],
        out_specs=pl.BlockSpec((blk, d), lambda i: (i, 0)),
        out_shape=jax.ShapeDtypeStruct((v, d), jnp.float32),
    )(lut)


def _gather_kernel_body(n_chunks, lut_hbm, idx_hbm, out_hbm,
                       idx_v, rows_v, sem):
    wid = lax.axis_index("s") * NC + lax.axis_index("c")
    pltpu.sync_copy(idx_hbm.at[wid], idx_v)

    def chunk_step(j, carry):
        pltpu.async_copy(lut_hbm.at[idx_v.at[j]], rows_v, sem).wait()
        pltpu.sync_copy(rows_v, out_hbm.at[wid, j])
        return carry

    lax.fori_loop(0, n_chunks, chunk_step, 0, unroll=False)


def _make_gather(n_chunks):
    mesh = plsc.VectorSubcoreMesh(core_axis_name="c", subcore_axis_name="s")
    return pl.kernel(
        functools.partial(_gather_kernel_body, n_chunks),
        out_type=jax.ShapeDtypeStruct((NW, n_chunks, CHUNK, D_MODEL),
                                      jnp.float32),
        mesh=mesh,
        scratch_types=[
            pltpu.VMEM((n_chunks, CHUNK), jnp.int32),
            pltpu.VMEM((CHUNK, D_MODEL), jnp.float32),
            pltpu.SemaphoreType.DMA,
        ],
    )


def kernel(x, lut):
    b, t = x.shape
    total = b * t
    assert total % (NW * CHUNK) == 0
    n_chunks = total // (NW * CHUNK)

    scaled = _scale_table(lut)
    idx = x.reshape(NW, n_chunks, CHUNK).astype(jnp.int32)
    out = _make_gather(n_chunks)(scaled, idx)
    return out.reshape(b, t, D_MODEL)
